# baseline (speedup 1.0000x reference)
import functools

import jax
import jax.numpy as jnp
from jax import lax
from jax.experimental import pallas as pl
from jax.experimental.pallas import tpu as pltpu
from jax.experimental.pallas import tpu_sc as plsc

_N = 16384
_G = 100000
_R = 3
_NC = 2
_NS = 16
_NW = _NC * _NS
_B = _N // _NW

_mesh = plsc.VectorSubcoreMesh(core_axis_name="c", subcore_axis_name="s")


@functools.partial(
    pl.kernel,
    mesh=_mesh,
    compiler_params=pltpu.CompilerParams(
        needs_layout_passes=False, use_tc_tiling_on_sc=False
    ),
    out_type=jax.ShapeDtypeStruct((_R, _N), jnp.float32),
    scratch_types=[
        pltpu.VMEM((_B,), jnp.int32),
        pltpu.VMEM((_B,), jnp.float32),
        pltpu.VMEM((_B,), jnp.float32),
        pltpu.VMEM((_B,), jnp.float32),
        pltpu.SemaphoreType.DMA,
        pltpu.SemaphoreType.DMA,
        pltpu.SemaphoreType.DMA,
    ],
)
def _gather_cols(idx_hbm, table_hbm, out_hbm, idx_v, r0, r1, r2, isem, sem, osem):
    wid = lax.axis_index("s") * _NC + lax.axis_index("c")
    base = wid * _B
    _H = _B // 2
    i0 = pltpu.async_copy(idx_hbm.at[pl.ds(base, _H)], idx_v.at[pl.ds(0, _H)], isem)
    i1 = pltpu.async_copy(
        idx_hbm.at[pl.ds(base + _H, _H)], idx_v.at[pl.ds(_H, _H)], isem
    )
    gathers = []
    i0.wait()
    for r, buf in ((0, r0), (1, r1), (2, r2)):
        gathers.append(pltpu.async_copy(
            table_hbm.at[r].at[idx_v.at[pl.ds(0, _H)]], buf.at[pl.ds(0, _H)], sem
        ))
    i1.wait()
    for r, buf in ((0, r0), (1, r1), (2, r2)):
        gathers.append(pltpu.async_copy(
            table_hbm.at[r].at[idx_v.at[pl.ds(_H, _H)]], buf.at[pl.ds(_H, _H)], sem
        ))
    stores = []
    for g, (r, buf) in zip(gathers, ((0, r0), (1, r1), (2, r2))):
        g.wait()
    for g in gathers[3:]:
        g.wait()
    for r, buf in ((0, r0), (1, r1), (2, r2)):
        stores.append(pltpu.async_copy(buf, out_hbm.at[r, pl.ds(base, _B)], osem))
    for s in stores:
        s.wait()


def kernel(gene_index_tensor_n, cell_index_tensor_n, downsampling_rate_tensor_n,
           total_obs_reads_per_cell_tensor_n, cell_features_nf,
           global_prior_params_gr):
    table_t = global_prior_params_gr.T
    out_t = _gather_cols(gene_index_tensor_n, table_t)
    return out_t.T


# speedup vs baseline: 1.0187x; 1.0187x over previous
import functools

import jax
import jax.numpy as jnp
from jax import lax
from jax.experimental import pallas as pl
from jax.experimental.pallas import tpu as pltpu
from jax.experimental.pallas import tpu_sc as plsc

_N = 16384
_G = 100000
_R = 3
_NC = 2
_NS = 16
_NW = _NC * _NS
_B = _N // _NW

_mesh = plsc.VectorSubcoreMesh(core_axis_name="c", subcore_axis_name="s")


@functools.partial(
    pl.kernel,
    mesh=_mesh,
    compiler_params=pltpu.CompilerParams(
        needs_layout_passes=False, use_tc_tiling_on_sc=False
    ),
    out_type=jax.ShapeDtypeStruct((_R, _N), jnp.float32),
    scratch_types=[
        pltpu.VMEM((_B,), jnp.int32),
        pltpu.VMEM((_B,), jnp.float32),
        pltpu.VMEM((_B,), jnp.float32),
        pltpu.VMEM((_B,), jnp.float32),
        pltpu.SemaphoreType.DMA,
        pltpu.SemaphoreType.DMA,
        pltpu.SemaphoreType.DMA,
        pltpu.SemaphoreType.DMA,
    ],
)
def _gather_cols(idx_hbm, table_hbm, out_hbm, idx_v, r0, r1, r2, g0s, g1s, g2s,
                 osem):
    wid = lax.axis_index("s") * _NC + lax.axis_index("c")
    base = wid * _B
    pltpu.sync_copy(idx_hbm.at[pl.ds(base, _B)], idx_v)
    g0 = pltpu.async_copy(table_hbm.at[0].at[idx_v], r0, g0s)
    g1 = pltpu.async_copy(table_hbm.at[1].at[idx_v], r1, g1s)
    g2 = pltpu.async_copy(table_hbm.at[2].at[idx_v], r2, g2s)
    g0.wait()
    s0 = pltpu.async_copy(r0, out_hbm.at[0, pl.ds(base, _B)], osem)
    g1.wait()
    s1 = pltpu.async_copy(r1, out_hbm.at[1, pl.ds(base, _B)], osem)
    g2.wait()
    s2 = pltpu.async_copy(r2, out_hbm.at[2, pl.ds(base, _B)], osem)
    s0.wait()
    s1.wait()
    s2.wait()


def kernel(gene_index_tensor_n, cell_index_tensor_n, downsampling_rate_tensor_n,
           total_obs_reads_per_cell_tensor_n, cell_features_nf,
           global_prior_params_gr):
    table_t = global_prior_params_gr.T
    out_t = _gather_cols(gene_index_tensor_n, table_t)
    return out_t.T
